# speedup vs baseline: 3.4230x; 3.4230x over previous
import jax
import jax.numpy as jnp
from jax.experimental import pallas as pl
import jax.experimental.pallas.tpu as pltpu

_NUM_IDX = 1000
_PAD = 1024


def _dd_kernel(sigma_ref, w_ref, sigmas_ref, x_ref, o_ref):
    b = pl.program_id(0)
    sig = sigma_ref[b]
    sp = sigmas_ref[:, :]
    dist = jnp.abs(sig - sp)
    minval = jnp.min(dist)
    lin = (jax.lax.broadcasted_iota(jnp.int32, (8, 128), 0) * 128
           + jax.lax.broadcasted_iota(jnp.int32, (8, 128), 1))
    idx = jnp.min(jnp.where(dist == minval, lin, jnp.int32(1 << 30)))
    sq = jnp.sum(jnp.where(lin == idx, sp, 0.0))
    a = -sq / jnp.sqrt(sq * sq + 1.0)
    bias = -sq * (idx.astype(jnp.float32) / _NUM_IDX)

    x0 = x_ref[0, 0]
    x1 = x_ref[0, 1]
    x2 = x_ref[0, 2]
    for d in range(3):
        w0 = w_ref[0, d]
        w1 = w_ref[1, d]
        w2 = w_ref[2, d]
        xd = (x0, x1, x2)[d]
        o_ref[0, d] = a * (w0 * x0 + w1 * x1 + w2 * x2) + xd + bias


@jax.jit
def kernel(inputs, sigma, W, sigmas):
    B, C, H, Wd = inputs.shape
    sigmas_p = jnp.concatenate(
        [sigmas, jnp.full((_PAD - _NUM_IDX,), 1e30, dtype=sigmas.dtype)]
    ).reshape(8, 128)
    return pl.pallas_call(
        _dd_kernel,
        grid=(B,),
        in_specs=[
            pl.BlockSpec(memory_space=pltpu.SMEM),
            pl.BlockSpec(memory_space=pltpu.SMEM),
            pl.BlockSpec((8, 128), lambda b: (0, 0)),
            pl.BlockSpec((1, C, H, Wd), lambda b: (b, 0, 0, 0)),
        ],
        out_specs=pl.BlockSpec((1, C, H, Wd), lambda b: (b, 0, 0, 0)),
        out_shape=jax.ShapeDtypeStruct((B, C, H, Wd), inputs.dtype),
        compiler_params=pltpu.CompilerParams(
            dimension_semantics=("arbitrary",),
        ),
    )(sigma, W, sigmas_p, inputs)


# speedup vs baseline: 3.9634x; 1.1579x over previous
import jax
import jax.numpy as jnp
from jax.experimental import pallas as pl
import jax.experimental.pallas.tpu as pltpu

_NUM_IDX = 1000
_PAD = 1024


def _dd_kernel(sigma_ref, w_ref, sigmas_ref, x_ref, o_ref):
    b = pl.program_id(0)
    sig = sigma_ref[b]
    sp = sigmas_ref[:, :]
    dist = jnp.abs(sig - sp)
    minval = jnp.min(dist)
    lin = (jax.lax.broadcasted_iota(jnp.int32, (8, 128), 0) * 128
           + jax.lax.broadcasted_iota(jnp.int32, (8, 128), 1))
    idx = jnp.min(jnp.where(dist == minval, lin, jnp.int32(1 << 30)))
    sq = jnp.sum(jnp.where(lin == idx, sp, 0.0))
    a = -sq / jnp.sqrt(sq * sq + 1.0)
    bias = -sq * (idx.astype(jnp.float32) / _NUM_IDX)

    aw = [[a * w_ref[c, d] for d in range(3)] for c in range(3)]
    ch = 16
    h = x_ref.shape[2]

    def body(i, carry):
        r = pl.multiple_of(i * ch, ch)
        x0 = x_ref[0, 0, pl.ds(r, ch), :]
        x1 = x_ref[0, 1, pl.ds(r, ch), :]
        x2 = x_ref[0, 2, pl.ds(r, ch), :]
        xs = (x0, x1, x2)
        for d in range(3):
            o_ref[0, d, pl.ds(r, ch), :] = (
                aw[0][d] * x0 + aw[1][d] * x1 + aw[2][d] * x2 + xs[d] + bias
            )
        return carry

    jax.lax.fori_loop(0, h // ch, body, 0)


@jax.jit
def kernel(inputs, sigma, W, sigmas):
    B, C, H, Wd = inputs.shape
    sigmas_p = jnp.concatenate(
        [sigmas, jnp.full((_PAD - _NUM_IDX,), 1e30, dtype=sigmas.dtype)]
    ).reshape(8, 128)
    return pl.pallas_call(
        _dd_kernel,
        grid=(B,),
        in_specs=[
            pl.BlockSpec(memory_space=pltpu.SMEM),
            pl.BlockSpec(memory_space=pltpu.SMEM),
            pl.BlockSpec((8, 128), lambda b: (0, 0)),
            pl.BlockSpec((1, C, H, Wd), lambda b: (b, 0, 0, 0)),
        ],
        out_specs=pl.BlockSpec((1, C, H, Wd), lambda b: (b, 0, 0, 0)),
        out_shape=jax.ShapeDtypeStruct((B, C, H, Wd), inputs.dtype),
        compiler_params=pltpu.CompilerParams(
            dimension_semantics=("arbitrary",),
        ),
    )(sigma, W, sigmas_p, inputs)
